# Initial kernel scaffold; baseline (speedup 1.0000x reference)
import jax
import jax.numpy as jnp
from jax.experimental import pallas as pl


def kernel(n_id, x0, edge_index, e_id, edge_type, node_type, local_node_idx, W_node0, node_att0, W_q0, W_k0, W_v0, W_self0, W_node1, node_att1, W_q1, W_k1, W_v1, W_self1):
    raise NotImplementedError("write your pallas kernel here")


# speedup vs baseline: 1.8273x; 1.8273x over previous
import functools

import jax
import jax.numpy as jnp
from jax.experimental import pallas as pl

N_NODES = 10000
IN_CH = 128
D = 128
NUM_REL = 4
NEG_SLOPE = 0.2

BN = 256
NPAD = 10240


def _k1_body(x_ref, wn_ref, wself_ref, attp_ref, h_ref, hs_ref, s_ref):
    h = jnp.dot(x_ref[...], wn_ref[...])
    h_ref[...] = h
    hs_ref[...] = jnp.dot(h, wself_ref[...])
    s_ref[...] = jnp.dot(h, attp_ref[...])


def _dense_pre(xp, Wn, Wself, attP):
    grid = (NPAD // BN,)
    return pl.pallas_call(
        _k1_body,
        grid=grid,
        in_specs=[
            pl.BlockSpec((BN, D), lambda i: (i, 0)),
            pl.BlockSpec((D, D), lambda i: (0, 0)),
            pl.BlockSpec((D, D), lambda i: (0, 0)),
            pl.BlockSpec((D, D), lambda i: (0, 0)),
        ],
        out_specs=[
            pl.BlockSpec((BN, D), lambda i: (i, 0)),
            pl.BlockSpec((BN, D), lambda i: (i, 0)),
            pl.BlockSpec((BN, D), lambda i: (i, 0)),
        ],
        out_shape=[
            jax.ShapeDtypeStruct((NPAD, D), jnp.float32),
            jax.ShapeDtypeStruct((NPAD, D), jnp.float32),
            jax.ShapeDtypeStruct((NPAD, D), jnp.float32),
        ],
    )(xp, Wn, Wself, attP)


def _k2_body(final, z_ref, hs_ref, wq_ref, wk_ref, wv_ref, out_ref):
    z = z_ref[...]
    hsb = hs_ref[...]
    ksum = jnp.zeros((BN, D), jnp.float32)
    vsum = jnp.zeros((BN, D), jnp.float32)
    for r in range(NUM_REL):
        ksum = ksum + jnp.dot(z[r], wk_ref[r])
        vsum = vsum + jnp.dot(z[r], wv_ref[r])
    out = jnp.zeros((BN, D), jnp.float32)
    for r in range(NUM_REL):
        q = jnp.dot(z[r], wq_ref[r])
        psi = jnp.sum(q * ksum, axis=-1)
        delta = psi[:, None] * vsum + hsb
        out = out + jax.nn.softmax(delta, axis=-1)
    if final:
        out_ref[...] = jax.nn.log_softmax(out, axis=-1)
    else:
        out_ref[...] = jnp.maximum(out, 0.0)


def _dense_post(zp, hs, Wq, Wk, Wv, final):
    grid = (NPAD // BN,)
    return pl.pallas_call(
        functools.partial(_k2_body, final),
        grid=grid,
        in_specs=[
            pl.BlockSpec((NUM_REL, BN, D), lambda i: (0, i, 0)),
            pl.BlockSpec((BN, D), lambda i: (i, 0)),
            pl.BlockSpec((NUM_REL, D, D), lambda i: (0, 0, 0)),
            pl.BlockSpec((NUM_REL, D, D), lambda i: (0, 0, 0)),
            pl.BlockSpec((NUM_REL, D, D), lambda i: (0, 0, 0)),
        ],
        out_specs=pl.BlockSpec((BN, D), lambda i: (i, 0)),
        out_shape=jax.ShapeDtypeStruct((NPAD, D), jnp.float32),
    )(zp, hs, Wq, Wk, Wv)


def _edge_phase(h, s, src, dst, et):
    n = N_NODES
    s_i = s[:n, :NUM_REL]
    s_j = s[:n, NUM_REL:2 * NUM_REL]
    alpha = jax.nn.leaky_relu(s_i[dst, et] + s_j[src, et], NEG_SLOPE)
    seg = et * n + dst
    amax = jax.ops.segment_max(alpha, seg, num_segments=NUM_REL * n)
    amax = jnp.where(jnp.isfinite(amax), amax, 0.0)
    ex = jnp.exp(alpha - amax[seg])
    denom = jax.ops.segment_sum(ex, seg, num_segments=NUM_REL * n)
    a = ex / (denom[seg] + 1e-16)
    zflat = jax.ops.segment_sum(a[:, None] * h[src], seg,
                                num_segments=NUM_REL * n)
    return zflat.reshape(NUM_REL, n, D)


def _layer(xp, src, dst, et, Wn, att, Wq, Wk, Wv, Wself, final):
    attP = jnp.zeros((D, D), jnp.float32)
    attP = attP.at[:, :NUM_REL].set(att[:, :D].T)
    attP = attP.at[:, NUM_REL:2 * NUM_REL].set(att[:, D:].T)
    h, hs, s = _dense_pre(xp, Wn, Wself, attP)
    z = _edge_phase(h, s, src, dst, et)
    zp = jnp.pad(z, ((0, 0), (0, NPAD - N_NODES), (0, 0)))
    return _dense_post(zp, hs, Wq, Wk, Wv, final)


def kernel(n_id, x0, edge_index, e_id, edge_type, node_type, local_node_idx,
           W_node0, node_att0, W_q0, W_k0, W_v0, W_self0,
           W_node1, node_att1, W_q1, W_k1, W_v1, W_self1):
    src = edge_index[0]
    dst = edge_index[1]
    et = edge_type
    xp = jnp.pad(x0, ((0, NPAD - N_NODES), (0, 0)))
    x1 = _layer(xp, src, dst, et, W_node0, node_att0, W_q0, W_k0, W_v0,
                W_self0, final=False)
    x2 = _layer(x1, src, dst, et, W_node1, node_att1, W_q1, W_k1, W_v1,
                W_self1, final=True)
    return x2[:N_NODES]


# speedup vs baseline: 24.1369x; 13.2094x over previous
import functools

import jax
import jax.numpy as jnp
from jax import lax
from jax.experimental import pallas as pl
from jax.experimental.pallas import tpu as pltpu
from jax.experimental.pallas import tpu_sc as plsc

N_NODES = 10000
D = 128
NUM_REL = 4
NEG = 0.2
E = 320000

BN = 256
NPAD = 10240

NW = 32
EPT = E // NW
ECH = 2000
NECH = EPT // ECH
ECHA = 400
NECHA = EPT // ECHA
NCHUNK = 5
CH = 2048
SEGC = CH * NUM_REL
NDEN = NPAD * NUM_REL
CAP = NPAD
RB = 64
NSUB = 16

_mesh = plsc.VectorSubcoreMesh(core_axis_name="c", subcore_axis_name="s")
_SC_PARAMS = pltpu.CompilerParams(needs_layout_passes=False)


def _wid():
    return lax.axis_index("s") * 2 + lax.axis_index("c")


def _io16():
    return lax.iota(jnp.int32, 16)


def _k1_body(x_ref, wn_ref, wself_ref, attp_ref, h_ref, hs_ref, s_ref):
    h = jnp.dot(x_ref[...], wn_ref[...])
    h_ref[...] = h
    hs_ref[...] = jnp.dot(h, wself_ref[...])
    s_ref[...] = jnp.dot(h, attp_ref[...])


def _dense_pre(xp, Wn, Wself, attP):
    return pl.pallas_call(
        _k1_body,
        grid=(NPAD // BN,),
        in_specs=[
            pl.BlockSpec((BN, D), lambda i: (i, 0)),
            pl.BlockSpec((D, D), lambda i: (0, 0)),
            pl.BlockSpec((D, D), lambda i: (0, 0)),
            pl.BlockSpec((D, D), lambda i: (0, 0)),
        ],
        out_specs=[
            pl.BlockSpec((BN, D), lambda i: (i, 0)),
            pl.BlockSpec((BN, D), lambda i: (i, 0)),
            pl.BlockSpec((BN, D), lambda i: (i, 0)),
        ],
        out_shape=[
            jax.ShapeDtypeStruct((NPAD, D), jnp.float32),
            jax.ShapeDtypeStruct((NPAD, D), jnp.float32),
            jax.ShapeDtypeStruct((NPAD, D), jnp.float32),
        ],
    )(xp, Wn, Wself, attP)


def _a1_body(si_h, sj_h, src_h, dst_h, et_h, mhat_h,
             ex_h, denp_h,
             si_v, sj_v, den_v, srcb, dstb, etb, exb, mhat_v):
    wid = _wid()
    pltpu.sync_copy(si_h, si_v)
    pltpu.sync_copy(sj_h, sj_v)
    pltpu.sync_copy(mhat_h, mhat_v)

    z16 = jnp.zeros((16,), jnp.float32)

    def zbody(i, _):
        den_v[pl.ds(i * 16, 16)] = z16
        return 0
    lax.fori_loop(0, NDEN // 16, zbody, 0)

    m = mhat_v[...]
    mr = [m[r] for r in range(NUM_REL)]

    base = wid * EPT
    for k in range(NECHA):
        off = base + k * ECHA
        pltpu.sync_copy(src_h.at[pl.ds(off, ECHA)], srcb)
        pltpu.sync_copy(dst_h.at[pl.ds(off, ECHA)], dstb)
        pltpu.sync_copy(et_h.at[pl.ds(off, ECHA)], etb)

        def gbody(g, _):
            sl = pl.ds(g * 16, 16)
            sg = srcb[sl]
            dg = dstb[sl]
            eg = etb[sl]
            ii = dg * 4 + eg
            ij = sg * 4 + eg
            siv = plsc.load_gather(si_v, [ii])
            sjv = plsc.load_gather(sj_v, [ij])
            al = siv + sjv
            al = jnp.where(al >= 0, al, al * NEG)
            mh = jnp.where(eg == 0, mr[0],
                           jnp.where(eg == 1, mr[1],
                                     jnp.where(eg == 2, mr[2], mr[3])))
            ex = jnp.exp(al - mh)
            exb[sl] = ex
            plsc.addupdate_scatter(den_v, [ii], ex)
            return 0
        lax.fori_loop(0, ECHA // 16, gbody, 0)
        pltpu.sync_copy(exb, ex_h.at[pl.ds(off, ECHA)])
    pltpu.sync_copy(den_v, denp_h.at[pl.ds(wid * NDEN, NDEN)])


def _a1(si, sj, src, dst, et, mhat16):
    f = functools.partial(
        pl.kernel, _a1_body,
        out_type=[jax.ShapeDtypeStruct((E,), jnp.float32),
                  jax.ShapeDtypeStruct((NW * NDEN,), jnp.float32)],
        mesh=_mesh,
        compiler_params=_SC_PARAMS,
        scratch_types=[
            pltpu.VMEM((NDEN,), jnp.float32),
            pltpu.VMEM((NDEN,), jnp.float32),
            pltpu.VMEM((NDEN,), jnp.float32),
            pltpu.VMEM((ECHA,), jnp.int32),
            pltpu.VMEM((ECHA,), jnp.int32),
            pltpu.VMEM((ECHA,), jnp.int32),
            pltpu.VMEM((ECHA,), jnp.float32),
            pltpu.VMEM((16,), jnp.float32),
        ],
    )
    return f()(si, sj, src, dst, et, mhat16)


_MW = NDEN // NW


def _m_body(denp_h, rden_h, acc_v, row_v):
    wid = _wid()
    z16 = jnp.zeros((16,), jnp.float32)

    def zbody(i, _):
        acc_v[pl.ds(i * 16, 16)] = z16
        return 0
    lax.fori_loop(0, _MW // 16, zbody, 0)

    for t in range(NW):
        pltpu.sync_copy(denp_h.at[pl.ds(t * NDEN + wid * _MW, _MW)], row_v)

        def abody(i, _):
            sl = pl.ds(i * 16, 16)
            acc_v[sl] = acc_v[sl] + row_v[sl]
            return 0
        lax.fori_loop(0, _MW // 16, abody, 0)

    def rbody(i, _):
        sl = pl.ds(i * 16, 16)
        acc_v[sl] = 1.0 / (acc_v[sl] + 1e-16)
        return 0
    lax.fori_loop(0, _MW // 16, rbody, 0)
    pltpu.sync_copy(acc_v, rden_h.at[pl.ds(wid * _MW, _MW)])


def _m(denp):
    f = functools.partial(
        pl.kernel, _m_body,
        out_type=[jax.ShapeDtypeStruct((NDEN,), jnp.float32)],
        mesh=_mesh,
        compiler_params=_SC_PARAMS,
        scratch_types=[
            pltpu.VMEM((_MW,), jnp.float32),
            pltpu.VMEM((_MW,), jnp.float32),
        ],
    )
    return f()(denp)[0]


def _a2_body(src_h, dst_h, et_h, ex_h, rden_h,
             rsrc_h, rseg_h, ra_h, cnt_h,
             rden_v, rsrc_v, rseg_v, ra_v, srcb, dstb, etb, exb, cnt_v):
    wid = _wid()
    pltpu.sync_copy(rden_h, rden_v)
    io = _io16()
    zi16 = jnp.zeros((16,), jnp.int32)
    zf16 = jnp.zeros((16,), jnp.float32)
    cnts = zi16
    base = wid * EPT
    for c in range(NCHUNK):
        def zbody(i, _):
            sl = pl.ds(i * 16, 16)
            rsrc_v[sl] = zi16
            rseg_v[sl] = zi16
            ra_v[sl] = zf16
            return 0
        lax.fori_loop(0, CAP // 16, zbody, 0)

        off = jnp.int32(0)
        for k in range(NECH):
            eoff = base + k * ECH
            pltpu.sync_copy(src_h.at[pl.ds(eoff, ECH)], srcb)
            pltpu.sync_copy(dst_h.at[pl.ds(eoff, ECH)], dstb)
            pltpu.sync_copy(et_h.at[pl.ds(eoff, ECH)], etb)
            pltpu.sync_copy(ex_h.at[pl.ds(eoff, ECH)], exb)

            def gbody(g, off):
                sl = pl.ds(g * 16, 16)
                sg = srcb[sl]
                dg = dstb[sl]
                eg = etb[sl]
                xg = exb[sl]
                msk = lax.shift_right_logical(dg, 11) == c
                ii = dg * 4 + eg
                a = xg * plsc.load_gather(rden_v, [ii])
                seg = (dg - c * CH) * 4 + eg
                plsc.store_compressed(rsrc_v.at[pl.ds(off, 16)], sg, mask=msk)
                plsc.store_compressed(rseg_v.at[pl.ds(off, 16)], seg, mask=msk)
                plsc.store_compressed(ra_v.at[pl.ds(off, 16)], a, mask=msk)
                pc = plsc.all_reduce_population_count(msk)
                pcs = pc if pc.ndim == 0 else pc[0]
                return off + pcs
            off = lax.fori_loop(0, ECH // 16, gbody, off)

        nb = (off + RB - 1) // RB
        cnts = jnp.where(io == c, nb, cnts)
        rbase = (wid * NCHUNK + c) * CAP
        pltpu.sync_copy(rsrc_v, rsrc_h.at[pl.ds(rbase, CAP)])
        pltpu.sync_copy(rseg_v, rseg_h.at[pl.ds(rbase, CAP)])
        pltpu.sync_copy(ra_v, ra_h.at[pl.ds(rbase, CAP)])
    cnt_v[...] = cnts
    pltpu.sync_copy(cnt_v, cnt_h.at[pl.ds(wid * 16, 16)])


def _a2(src, dst, et, ex, rden):
    f = functools.partial(
        pl.kernel, _a2_body,
        out_type=[jax.ShapeDtypeStruct((NW * NCHUNK * CAP,), jnp.int32),
                  jax.ShapeDtypeStruct((NW * NCHUNK * CAP,), jnp.int32),
                  jax.ShapeDtypeStruct((NW * NCHUNK * CAP,), jnp.float32),
                  jax.ShapeDtypeStruct((NW * 16,), jnp.int32)],
        mesh=_mesh,
        compiler_params=_SC_PARAMS,
        scratch_types=[
            pltpu.VMEM((NDEN,), jnp.float32),
            pltpu.VMEM((CAP,), jnp.int32),
            pltpu.VMEM((CAP,), jnp.int32),
            pltpu.VMEM((CAP,), jnp.float32),
            pltpu.VMEM((ECH,), jnp.int32),
            pltpu.VMEM((ECH,), jnp.int32),
            pltpu.VMEM((ECH,), jnp.int32),
            pltpu.VMEM((ECH,), jnp.float32),
            pltpu.VMEM((16,), jnp.int32),
        ],
    )
    return f()(src, dst, et, ex, rden)


def _b_body(h_h, rsrc_h, rseg_h, ra_h, cnt_h, zeros_h,
            zp_h,
            rows_v, srcb2, segb2, ab, cnt_v, z_sh):
    cidx = lax.axis_index("c")
    sidx = lax.axis_index("s")
    wid = sidx * 2 + cidx
    pltpu.sync_copy(cnt_h.at[pl.ds(wid * 16, 16)], cnt_v)
    cv = cnt_v[...]
    nbs = [cv[c] for c in range(NCHUNK)]
    myrows = SEGC // NSUB

    for c in range(NCHUNK):
        pltpu.sync_copy(zeros_h, z_sh.at[pl.ds(sidx * myrows, myrows)])
        plsc.subcore_barrier()

        rbase = (wid * NCHUNK + c) * CAP

        def bbody(b, _):
            pltpu.sync_copy(rsrc_h.at[pl.ds(rbase + b * RB, RB)], srcb2.at[0])
            pltpu.sync_copy(rseg_h.at[pl.ds(rbase + b * RB, RB)], segb2.at[0])
            pltpu.sync_copy(ra_h.at[pl.ds(rbase + b * RB, RB)], ab)
            pltpu.sync_copy(h_h.at[srcb2.at[0]], rows_v)
            for g in range(RB // 16):
                a16 = ab[pl.ds(g * 16, 16)]
                for l in range(16):
                    e = g * 16 + l
                    sc = a16[l]
                    for j in range(D // 16):
                        sl = pl.ds(j * 16, 16)
                        rows_v[e, sl] = rows_v[e, sl] * sc
            pltpu.sync_copy(rows_v, z_sh.at[segb2.at[0]], add=True)
            return 0
        lax.fori_loop(0, nbs[c], bbody, 0)
        plsc.subcore_barrier()
        pltpu.sync_copy(z_sh.at[pl.ds(sidx * myrows, myrows)],
                        zp_h.at[cidx, c, pl.ds(sidx * myrows, myrows)])


def _b(h, rsrc, rseg, ra, cnt, zeros512):
    f = functools.partial(
        pl.kernel, _b_body,
        out_type=[jax.ShapeDtypeStruct((2, NCHUNK, SEGC, D), jnp.float32)],
        mesh=_mesh,
        compiler_params=_SC_PARAMS,
        scratch_types=[
            pltpu.VMEM((RB, D), jnp.float32),
            pltpu.VMEM((1, RB), jnp.int32),
            pltpu.VMEM((1, RB), jnp.int32),
            pltpu.VMEM((RB,), jnp.float32),
            pltpu.VMEM((16,), jnp.int32),
            pltpu.VMEM_SHARED((SEGC, D), jnp.float32),
        ],
    )
    return f()(h, rsrc, rseg, ra, cnt, zeros512)[0]


def _k2_body(final, zp_ref, hs_ref, wq_ref, wk_ref, wv_ref, out_ref):
    zb = zp_ref[...]
    zb = zb.reshape(2, BN, NUM_REL, D)
    hsb = hs_ref[...]
    zs = [zb[0, :, r, :] + zb[1, :, r, :] for r in range(NUM_REL)]
    ksum = jnp.zeros((BN, D), jnp.float32)
    vsum = jnp.zeros((BN, D), jnp.float32)
    for r in range(NUM_REL):
        ksum = ksum + jnp.dot(zs[r], wk_ref[r])
        vsum = vsum + jnp.dot(zs[r], wv_ref[r])
    out = jnp.zeros((BN, D), jnp.float32)
    for r in range(NUM_REL):
        q = jnp.dot(zs[r], wq_ref[r])
        psi = jnp.sum(q * ksum, axis=-1)
        delta = psi[:, None] * vsum + hsb
        out = out + jax.nn.softmax(delta, axis=-1)
    if final:
        out_ref[...] = jax.nn.log_softmax(out, axis=-1)
    else:
        out_ref[...] = jnp.maximum(out, 0.0)


def _dense_post(zp, hs, Wq, Wk, Wv, final):
    nsub = CH // BN
    return pl.pallas_call(
        functools.partial(_k2_body, final),
        grid=(NPAD // BN,),
        in_specs=[
            pl.BlockSpec((2, 1, NUM_REL * BN, D),
                         lambda i: (0, i // 8, i % 8, 0)),
            pl.BlockSpec((BN, D), lambda i: (i, 0)),
            pl.BlockSpec((NUM_REL, D, D), lambda i: (0, 0, 0)),
            pl.BlockSpec((NUM_REL, D, D), lambda i: (0, 0, 0)),
            pl.BlockSpec((NUM_REL, D, D), lambda i: (0, 0, 0)),
        ],
        out_specs=pl.BlockSpec((BN, D), lambda i: (i, 0)),
        out_shape=jax.ShapeDtypeStruct((NPAD, D), jnp.float32),
    )(zp, hs, Wq, Wk, Wv)


def _layer(xp, src, dst, et, zeros512, Wn, att, Wq, Wk, Wv, Wself, final):
    attP = jnp.zeros((D, D), jnp.float32)
    attP = attP.at[:, :NUM_REL].set(att[:, :D].T)
    attP = attP.at[:, NUM_REL:2 * NUM_REL].set(att[:, D:].T)
    h, hs, s = _dense_pre(xp, Wn, Wself, attP)
    si = s[:, :NUM_REL].reshape(-1)
    sj = s[:, NUM_REL:2 * NUM_REL].reshape(-1)
    sreal = s[:N_NODES]
    mh = jnp.max(sreal[:, :NUM_REL], axis=0) + \
        jnp.max(sreal[:, NUM_REL:2 * NUM_REL], axis=0)
    mh = jnp.where(mh >= 0, mh, NEG * mh)
    mhat16 = jnp.zeros((16,), jnp.float32).at[:NUM_REL].set(mh)
    ex, denp = _a1(si, sj, src, dst, et, mhat16)
    rden = _m(denp)
    rsrc, rseg, ra, cnt = _a2(src, dst, et, ex, rden)
    zp = _b(h, rsrc, rseg, ra, cnt, zeros512)
    return _dense_post(zp, hs, Wq, Wk, Wv, final)


def kernel(n_id, x0, edge_index, e_id, edge_type, node_type, local_node_idx,
           W_node0, node_att0, W_q0, W_k0, W_v0, W_self0,
           W_node1, node_att1, W_q1, W_k1, W_v1, W_self1):
    src = edge_index[0]
    dst = edge_index[1]
    et = edge_type
    zeros512 = jnp.zeros((SEGC // NSUB, D), jnp.float32)
    xp = jnp.pad(x0, ((0, NPAD - N_NODES), (0, 0)))
    x1 = _layer(xp, src, dst, et, zeros512, W_node0, node_att0, W_q0, W_k0,
                W_v0, W_self0, final=False)
    x2 = _layer(x1, src, dst, et, zeros512, W_node1, node_att1, W_q1, W_k1,
                W_v1, W_self1, final=True)
    return x2[:N_NODES]
